# baseline (speedup 1.0000x reference)
import functools

import jax
import jax.numpy as jnp
import numpy as np
from jax.experimental import pallas as pl
from jax.experimental.pallas import tpu as pltpu

WSZ = 8
FILTER_RATE = 0.3
EPS = 1e-5


def _gelu(x):
    return x * 0.5 * (1.0 + jax.lax.erf(x * 0.7071067811865476))


def _scores_body(u_ref, poolT_ref, pool_ref, s_ref):
    inv = 1.0 / (WSZ * WSZ)
    hp = jax.lax.Precision.HIGHEST
    for b in range(u_ref.shape[0]):
        t = jax.lax.dot(poolT_ref[...], u_ref[b], precision=hp,
                        preferred_element_type=jnp.float32)
        s_ref[b] = jax.lax.dot(t, pool_ref[...], precision=hp,
                               preferred_element_type=jnp.float32) * inv


def _refine_body(m_ref, x_ref, nw_ref, nb_ref, l0w_ref, l0b_ref,
                 qw_ref, pw_ref, pb_ref, o_ref, att_ref,
                 *, C, C2, nwin_blk, scale):
    winsz = WSZ * WSZ
    SP = nwin_blk * winsz

    X3 = x_ref[0]
    Xw = jnp.swapaxes(X3, 1, 2).reshape(C, SP)

    mu = jnp.mean(Xw, axis=0, keepdims=True)
    var = jnp.mean((Xw - mu) ** 2, axis=0, keepdims=True)
    Xn = (Xw - mu) * jax.lax.rsqrt(var + EPS) * nw_ref[...] + nb_ref[...]

    h = _gelu(jax.lax.dot(l0w_ref[...], Xn,
                          preferred_element_type=jnp.float32) + l0b_ref[...])
    qkv = jax.lax.dot(qw_ref[...], h, preferred_element_type=jnp.float32)

    dn = (((0,), (0,)), ((), ()))
    for w in range(nwin_blk):
        sl = slice(w * winsz, (w + 1) * winsz)
        qw_ = qkv[0:C2, sl]
        kw_ = qkv[C2:2 * C2, sl]
        vw_ = qkv[2 * C2:3 * C2, sl]
        S = jax.lax.dot_general(kw_, qw_, dn,
                                preferred_element_type=jnp.float32) * scale
        S = S - jnp.max(S, axis=0, keepdims=True)
        E = jnp.exp(S)
        P = E / jnp.sum(E, axis=0, keepdims=True)
        att_ref[:, sl] = jax.lax.dot(vw_, P,
                                     preferred_element_type=jnp.float32)

    xf = h + att_ref[...]
    delta = _gelu(jax.lax.dot(pw_ref[...], xf,
                              preferred_element_type=jnp.float32) + pb_ref[...])
    delta = (delta + Xn) * m_ref[0, 0]
    D3 = jnp.swapaxes(delta.reshape(C, SP // WSZ, WSZ), 1, 2)
    o_ref[0] = X3 + D3


def kernel(feature_map, uncertain_map, norm_w, norm_b, lin0_W, lin0_b,
           qkv0_W, proj_W, proj_b):
    B, C, H, W = feature_map.shape
    nH, nW = H // WSZ, W // WSZ
    nWin = nH * nW
    winsz = WSZ * WSZ
    nWF = int(nWin * FILTER_RATE)
    C2 = lin0_W.shape[1]
    scale = float(C ** (-0.5))

    pool = (np.arange(W)[:, None] // WSZ == np.arange(nW)[None, :])
    pool = jnp.asarray(pool, dtype=jnp.float32)
    poolT = (np.arange(nH)[:, None] == np.arange(H)[None, :] // WSZ)
    poolT = jnp.asarray(poolT, dtype=jnp.float32)
    scores = pl.pallas_call(
        _scores_body,
        out_shape=jax.ShapeDtypeStruct((B, nH, nW), jnp.float32),
    )(uncertain_map, poolT, pool)

    win_score = scores.reshape(B, nWin)
    _, idx = jax.lax.top_k(win_score, nWF)
    mask = jnp.zeros((B, nWin), jnp.float32)
    mask = mask.at[jnp.arange(B)[:, None], idx].set(1.0)
    maskp = jnp.repeat(mask.reshape(B, nH, nW), winsz, axis=-1)
    maskp = maskp.reshape(B, nH, 1, nW * winsz)

    G = 4
    LB = W // G
    nwin_blk = LB // WSZ
    SP = nwin_blk * winsz

    l0wt = lin0_W.T
    qwt = qkv0_W.T
    pwt = proj_W.T
    l0b2 = lin0_b.reshape(C2, 1)
    pb2 = proj_b.reshape(C, 1)
    nw2 = norm_w.reshape(C, 1)
    nb2 = norm_b.reshape(C, 1)

    body = functools.partial(_refine_body, C=C, C2=C2, nwin_blk=nwin_blk,
                             scale=scale)
    out = pl.pallas_call(
        body,
        grid=(B, nH, G),
        in_specs=[
            pl.BlockSpec((1, 1, 1, SP), lambda b, s, g: (b, s, 0, g)),
            pl.BlockSpec((1, C, WSZ, LB), lambda b, s, g: (b, 0, s, g)),
            pl.BlockSpec((C, 1), lambda b, s, g: (0, 0)),
            pl.BlockSpec((C, 1), lambda b, s, g: (0, 0)),
            pl.BlockSpec((C2, C), lambda b, s, g: (0, 0)),
            pl.BlockSpec((C2, 1), lambda b, s, g: (0, 0)),
            pl.BlockSpec((3 * C2, C2), lambda b, s, g: (0, 0)),
            pl.BlockSpec((C, C2), lambda b, s, g: (0, 0)),
            pl.BlockSpec((C, 1), lambda b, s, g: (0, 0)),
        ],
        out_specs=pl.BlockSpec((1, C, WSZ, LB), lambda b, s, g: (b, 0, s, g)),
        out_shape=jax.ShapeDtypeStruct((B, C, H, W), jnp.float32),
        scratch_shapes=[pltpu.VMEM((C2, SP), jnp.float32)],
    )(maskp, feature_map, nw2, nb2, l0wt, l0b2, qwt, pwt, pb2)
    return out


# speedup vs baseline: 3.3215x; 3.3215x over previous
import functools

import jax
import jax.numpy as jnp
import numpy as np
from jax.experimental import pallas as pl
from jax.experimental.pallas import tpu as pltpu

WSZ = 8
FILTER_RATE = 0.3
EPS = 1e-5


def _gelu(x):
    return x * 0.5 * (1.0 + jax.lax.erf(x * 0.7071067811865476))


def _scores_body(u_ref, poolT_ref, pool_ref, s_ref):
    inv = 1.0 / (WSZ * WSZ)
    hp = jax.lax.Precision.HIGHEST
    for b in range(u_ref.shape[0]):
        t = jax.lax.dot(poolT_ref[...], u_ref[b], precision=hp,
                        preferred_element_type=jnp.float32)
        s_ref[b] = jax.lax.dot(t, pool_ref[...], precision=hp,
                               preferred_element_type=jnp.float32) * inv


def _refine_body(m_ref, x_ref, e_ref, et_ref, nw_ref, nb_ref, l0w_ref,
                 l0b_ref, qw_ref, pw_ref, pb_ref, o_ref, att_ref,
                 *, C, C2, LB, scale):
    winsz = WSZ * WSZ
    nwin = LB // WSZ
    SP = LB * WSZ

    X3 = x_ref[0]

    mu = jnp.mean(X3, axis=0, keepdims=True)
    var = jnp.mean((X3 - mu) ** 2, axis=0, keepdims=True)
    Xn3 = (X3 - mu) * jax.lax.rsqrt(var + EPS) * nw_ref[...].reshape(C, 1, 1) \
        + nb_ref[...].reshape(C, 1, 1)
    Xn3b = Xn3.astype(jnp.bfloat16)

    Xn = jnp.zeros((C, SP), jnp.float32)
    for r in range(WSZ):
        Xn = Xn + jax.lax.dot(Xn3b[:, r, :], e_ref[r * LB:(r + 1) * LB, :],
                              preferred_element_type=jnp.float32)
    Xnb = Xn.astype(jnp.bfloat16)

    h = _gelu(jax.lax.dot(l0w_ref[...], Xnb,
                          preferred_element_type=jnp.float32) + l0b_ref[...])
    hb = h.astype(jnp.bfloat16)
    qkv = jax.lax.dot(qw_ref[...], hb,
                      preferred_element_type=jnp.float32).astype(jnp.bfloat16)

    dn = (((0,), (0,)), ((), ()))
    for w in range(nwin):
        sl = slice(w * winsz, (w + 1) * winsz)
        qw_ = qkv[0:C2, sl]
        kw_ = qkv[C2:2 * C2, sl]
        vw_ = qkv[2 * C2:3 * C2, sl]
        S = jax.lax.dot_general(kw_, qw_, dn,
                                preferred_element_type=jnp.float32) * scale
        S = S - jnp.max(S, axis=0, keepdims=True)
        E = jnp.exp(S)
        P = (E / jnp.sum(E, axis=0, keepdims=True)).astype(jnp.bfloat16)
        att_ref[:, sl] = jax.lax.dot(vw_, P,
                                     preferred_element_type=jnp.float32)

    xf = (h + att_ref[...]).astype(jnp.bfloat16)
    delta = _gelu(jax.lax.dot(pw_ref[...], xf,
                              preferred_element_type=jnp.float32) + pb_ref[...])
    delta = ((delta + Xn) * m_ref[0, 0]).astype(jnp.bfloat16)
    for r in range(WSZ):
        d_r = jax.lax.dot(delta, et_ref[:, r * LB:(r + 1) * LB],
                          preferred_element_type=jnp.float32)
        o_ref[0, :, r, :] = X3[:, r, :] + d_r


def kernel(feature_map, uncertain_map, norm_w, norm_b, lin0_W, lin0_b,
           qkv0_W, proj_W, proj_b):
    B, C, H, W = feature_map.shape
    nH, nW = H // WSZ, W // WSZ
    nWin = nH * nW
    winsz = WSZ * WSZ
    nWF = int(nWin * FILTER_RATE)
    C2 = lin0_W.shape[1]
    scale = float(C ** (-0.5))

    pool = (np.arange(W)[:, None] // WSZ == np.arange(nW)[None, :])
    pool = jnp.asarray(pool, dtype=jnp.float32)
    poolT = (np.arange(nH)[:, None] == np.arange(H)[None, :] // WSZ)
    poolT = jnp.asarray(poolT, dtype=jnp.float32)
    scores = pl.pallas_call(
        _scores_body,
        out_shape=jax.ShapeDtypeStruct((B, nH, nW), jnp.float32),
    )(uncertain_map, poolT, pool)

    win_score = scores.reshape(B, nWin)
    _, idx = jax.lax.top_k(win_score, nWF)
    mask = jnp.zeros((B, nWin), jnp.float32)
    mask = mask.at[jnp.arange(B)[:, None], idx].set(1.0)
    maskp = jnp.repeat(mask.reshape(B, nH, nW), winsz, axis=-1)
    maskp = maskp.reshape(B, nH, 1, nW * winsz)

    G = 4
    LB = W // G
    SP = LB * WSZ

    l_ = np.arange(LB)
    Emat = np.zeros((WSZ * LB, SP), np.float32)
    for r in range(WSZ):
        Emat[r * LB + l_, WSZ * l_ + r] = 1.0
    Eb = jnp.asarray(Emat, dtype=jnp.bfloat16)
    ETb = jnp.asarray(Emat.T.copy(), dtype=jnp.bfloat16)

    l0wt = jnp.asarray(lin0_W.T, dtype=jnp.bfloat16)
    qwt = jnp.asarray(qkv0_W.T, dtype=jnp.bfloat16)
    pwt = jnp.asarray(proj_W.T, dtype=jnp.bfloat16)
    l0b2 = lin0_b.reshape(C2, 1)
    pb2 = proj_b.reshape(C, 1)
    nw2 = norm_w.reshape(C, 1)
    nb2 = norm_b.reshape(C, 1)

    body = functools.partial(_refine_body, C=C, C2=C2, LB=LB, scale=scale)
    out = pl.pallas_call(
        body,
        grid=(B, nH, G),
        in_specs=[
            pl.BlockSpec((1, 1, 1, SP), lambda b, s, g: (b, s, 0, g)),
            pl.BlockSpec((1, C, WSZ, LB), lambda b, s, g: (b, 0, s, g)),
            pl.BlockSpec((WSZ * LB, SP), lambda b, s, g: (0, 0)),
            pl.BlockSpec((SP, WSZ * LB), lambda b, s, g: (0, 0)),
            pl.BlockSpec((C, 1), lambda b, s, g: (0, 0)),
            pl.BlockSpec((C, 1), lambda b, s, g: (0, 0)),
            pl.BlockSpec((C2, C), lambda b, s, g: (0, 0)),
            pl.BlockSpec((C2, 1), lambda b, s, g: (0, 0)),
            pl.BlockSpec((3 * C2, C2), lambda b, s, g: (0, 0)),
            pl.BlockSpec((C, C2), lambda b, s, g: (0, 0)),
            pl.BlockSpec((C, 1), lambda b, s, g: (0, 0)),
        ],
        out_specs=pl.BlockSpec((1, C, WSZ, LB), lambda b, s, g: (b, 0, s, g)),
        out_shape=jax.ShapeDtypeStruct((B, C, H, W), jnp.float32),
        scratch_shapes=[pltpu.VMEM((C2, SP), jnp.float32)],
    )(maskp, feature_map, Eb, ETb, nw2, nb2, l0wt, l0b2, qwt, pwt, pb2)
    return out


# speedup vs baseline: 3.9459x; 1.1880x over previous
import functools

import jax
import jax.numpy as jnp
import numpy as np
from jax.experimental import pallas as pl
from jax.experimental.pallas import tpu as pltpu

WSZ = 8
FILTER_RATE = 0.3
EPS = 1e-5


def _gelu(x):
    return x * 0.5 * (1.0 + jax.lax.erf(x * 0.7071067811865476))


def _scores_body(u_ref, poolT_ref, pool_ref, s_ref):
    inv = 1.0 / (WSZ * WSZ)
    hp = jax.lax.Precision.HIGHEST
    for b in range(u_ref.shape[0]):
        t = jax.lax.dot(poolT_ref[...], u_ref[b], precision=hp,
                        preferred_element_type=jnp.float32)
        s_ref[b] = jax.lax.dot(t, pool_ref[...], precision=hp,
                               preferred_element_type=jnp.float32) * inv


def _refine_body(m_ref, x_ref, e_ref, et_ref, nw_ref, nb_ref, l0w_ref,
                 l0b_ref, qw_ref, pw_ref, pb_ref, o_ref,
                 h_ref, qkv_ref, att_ref, *, C, C2, LB, scale):
    winsz = WSZ * WSZ
    nwin = LB // WSZ
    SP = LB * WSZ

    X3 = x_ref[0]
    X3b = X3.astype(jnp.bfloat16)

    Xw = jnp.zeros((C, SP), jnp.float32)
    for r in range(WSZ):
        Xw = Xw + jax.lax.dot(X3b[:, r, :], e_ref[r * LB:(r + 1) * LB, :],
                              preferred_element_type=jnp.float32)

    mu = jnp.mean(Xw, axis=0, keepdims=True)
    var = jnp.mean(Xw * Xw, axis=0, keepdims=True) - mu * mu
    Xnb = ((Xw - mu) * jax.lax.rsqrt(var + EPS) * nw_ref[...]
           + nb_ref[...]).astype(jnp.bfloat16)

    h_ref[...] = _gelu(
        jax.lax.dot(l0w_ref[...], Xnb, preferred_element_type=jnp.float32)
        + l0b_ref[...]).astype(jnp.bfloat16)
    qkv_ref[...] = jax.lax.dot(
        qw_ref[...], h_ref[...],
        preferred_element_type=jnp.float32).astype(jnp.bfloat16)

    dn = (((0,), (0,)), ((), ()))
    for w in range(nwin):
        sl = slice(w * winsz, (w + 1) * winsz)
        qw_ = qkv_ref[0:C2, sl]
        kw_ = qkv_ref[C2:2 * C2, sl]
        vw_ = qkv_ref[2 * C2:3 * C2, sl]
        S = jax.lax.dot_general(kw_, qw_, dn,
                                preferred_element_type=jnp.float32) * scale
        S = S - jnp.max(S, axis=0, keepdims=True)
        E = jnp.exp(S)
        P = (E / jnp.sum(E, axis=0, keepdims=True)).astype(jnp.bfloat16)
        att_ref[:, sl] = jax.lax.dot(
            vw_, P, preferred_element_type=jnp.float32).astype(jnp.bfloat16)

    xfb = h_ref[...] + att_ref[...]
    delta = _gelu(jax.lax.dot(pw_ref[...], xfb,
                              preferred_element_type=jnp.float32) + pb_ref[...])
    deltab = delta.astype(jnp.bfloat16) + Xnb
    D = jax.lax.dot(deltab, et_ref[...], preferred_element_type=jnp.float32)
    ml = m_ref[0, 0]
    for r in range(WSZ):
        o_ref[0, :, r, :] = X3[:, r, :] + D[:, r * LB:(r + 1) * LB] * ml


def kernel(feature_map, uncertain_map, norm_w, norm_b, lin0_W, lin0_b,
           qkv0_W, proj_W, proj_b):
    B, C, H, W = feature_map.shape
    nH, nW = H // WSZ, W // WSZ
    nWin = nH * nW
    winsz = WSZ * WSZ
    nWF = int(nWin * FILTER_RATE)
    C2 = lin0_W.shape[1]
    scale = float(C ** (-0.5))

    pool = (np.arange(W)[:, None] // WSZ == np.arange(nW)[None, :])
    pool = jnp.asarray(pool, dtype=jnp.float32)
    poolT = (np.arange(nH)[:, None] == np.arange(H)[None, :] // WSZ)
    poolT = jnp.asarray(poolT, dtype=jnp.float32)
    scores = pl.pallas_call(
        _scores_body,
        out_shape=jax.ShapeDtypeStruct((B, nH, nW), jnp.float32),
    )(uncertain_map, poolT, pool)

    win_score = scores.reshape(B, nWin)
    _, idx = jax.lax.top_k(win_score, nWF)
    mask = jnp.zeros((B, nWin), jnp.float32)
    mask = mask.at[jnp.arange(B)[:, None], idx].set(1.0)
    maskl = jnp.repeat(mask.reshape(B, nH, nW), WSZ, axis=-1)
    maskl = maskl.reshape(B, nH, 1, W)

    G = 4
    LB = W // G
    SP = LB * WSZ

    l_ = np.arange(LB)
    Emat = np.zeros((WSZ * LB, SP), np.float32)
    for r in range(WSZ):
        Emat[r * LB + l_, WSZ * l_ + r] = 1.0
    Eb = jnp.asarray(Emat, dtype=jnp.bfloat16)
    ETb = jnp.asarray(Emat.T.copy(), dtype=jnp.bfloat16)

    l0wt = jnp.asarray(lin0_W.T, dtype=jnp.bfloat16)
    qwt = jnp.asarray(qkv0_W.T, dtype=jnp.bfloat16)
    pwt = jnp.asarray(proj_W.T, dtype=jnp.bfloat16)
    l0b2 = lin0_b.reshape(C2, 1)
    pb2 = proj_b.reshape(C, 1)
    nw2 = norm_w.reshape(C, 1)
    nb2 = norm_b.reshape(C, 1)

    body = functools.partial(_refine_body, C=C, C2=C2, LB=LB, scale=scale)
    out = pl.pallas_call(
        body,
        grid=(B, nH, G),
        in_specs=[
            pl.BlockSpec((1, 1, 1, LB), lambda b, s, g: (b, s, 0, g)),
            pl.BlockSpec((1, C, WSZ, LB), lambda b, s, g: (b, 0, s, g)),
            pl.BlockSpec((WSZ * LB, SP), lambda b, s, g: (0, 0)),
            pl.BlockSpec((SP, WSZ * LB), lambda b, s, g: (0, 0)),
            pl.BlockSpec((C, 1), lambda b, s, g: (0, 0)),
            pl.BlockSpec((C, 1), lambda b, s, g: (0, 0)),
            pl.BlockSpec((C2, C), lambda b, s, g: (0, 0)),
            pl.BlockSpec((C2, 1), lambda b, s, g: (0, 0)),
            pl.BlockSpec((3 * C2, C2), lambda b, s, g: (0, 0)),
            pl.BlockSpec((C, C2), lambda b, s, g: (0, 0)),
            pl.BlockSpec((C, 1), lambda b, s, g: (0, 0)),
        ],
        out_specs=pl.BlockSpec((1, C, WSZ, LB), lambda b, s, g: (b, 0, s, g)),
        out_shape=jax.ShapeDtypeStruct((B, C, H, W), jnp.float32),
        scratch_shapes=[pltpu.VMEM((C2, SP), jnp.bfloat16),
                        pltpu.VMEM((3 * C2, SP), jnp.bfloat16),
                        pltpu.VMEM((C2, SP), jnp.bfloat16)],
    )(maskl, feature_map, Eb, ETb, nw2, nb2, l0wt, l0b2, qwt, pwt, pb2)
    return out


# speedup vs baseline: 4.0657x; 1.0304x over previous
import functools

import jax
import jax.numpy as jnp
import numpy as np
from jax.experimental import pallas as pl
from jax.experimental.pallas import tpu as pltpu

WSZ = 8
FILTER_RATE = 0.3
EPS = 1e-5


def _gelu(x):
    return x * 0.5 * (1.0 + jax.lax.erf(x * 0.7071067811865476))


def _scores_body(u_ref, poolT_ref, pool_ref, s_ref):
    inv = 1.0 / (WSZ * WSZ)
    hp = jax.lax.Precision.HIGHEST
    for b in range(u_ref.shape[0]):
        t = jax.lax.dot(poolT_ref[...], u_ref[b], precision=hp,
                        preferred_element_type=jnp.float32)
        s_ref[b] = jax.lax.dot(t, pool_ref[...], precision=hp,
                               preferred_element_type=jnp.float32) * inv


def _refine_body(m_ref, x_ref, e_ref, et_ref, nw_ref, nb_ref, l0w_ref,
                 l0b_ref, qw_ref, pw_ref, pb_ref, o_ref,
                 h_ref, qkv_ref, att_ref, *, C, C2, LB, NS, scale):
    for s in range(NS):
        _refine_strip(m_ref, x_ref, e_ref, et_ref, nw_ref, nb_ref, l0w_ref,
                      l0b_ref, qw_ref, pw_ref, pb_ref, o_ref,
                      h_ref, qkv_ref, att_ref, s, C=C, C2=C2, LB=LB,
                      scale=scale)


def _refine_strip(m_ref, x_ref, e_ref, et_ref, nw_ref, nb_ref, l0w_ref,
                  l0b_ref, qw_ref, pw_ref, pb_ref, o_ref,
                  h_ref, qkv_ref, att_ref, s, *, C, C2, LB, scale):
    winsz = WSZ * WSZ
    nwin = LB // WSZ
    SP = LB * WSZ

    X3 = x_ref[0, :, s * WSZ:(s + 1) * WSZ, :]
    X3b = X3.astype(jnp.bfloat16)

    Xw = jnp.zeros((C, SP), jnp.float32)
    for r in range(WSZ):
        Xw = Xw + jax.lax.dot(X3b[:, r, :], e_ref[r * LB:(r + 1) * LB, :],
                              preferred_element_type=jnp.float32)

    mu = jnp.mean(Xw, axis=0, keepdims=True)
    var = jnp.mean(Xw * Xw, axis=0, keepdims=True) - mu * mu
    Xnb = ((Xw - mu) * jax.lax.rsqrt(var + EPS) * nw_ref[...]
           + nb_ref[...]).astype(jnp.bfloat16)

    h_ref[...] = _gelu(
        jax.lax.dot(l0w_ref[...], Xnb, preferred_element_type=jnp.float32)
        + l0b_ref[...]).astype(jnp.bfloat16)
    qkv_ref[...] = jax.lax.dot(
        qw_ref[...], h_ref[...],
        preferred_element_type=jnp.float32).astype(jnp.bfloat16)

    dn = (((0,), (0,)), ((), ()))
    for w in range(nwin):
        sl = slice(w * winsz, (w + 1) * winsz)
        qw_ = qkv_ref[0:C2, sl]
        kw_ = qkv_ref[C2:2 * C2, sl]
        vw_ = qkv_ref[2 * C2:3 * C2, sl]
        S = jax.lax.dot_general(kw_, qw_, dn,
                                preferred_element_type=jnp.float32) * scale
        S = S - jnp.max(S, axis=0, keepdims=True)
        E = jnp.exp(S)
        P = (E / jnp.sum(E, axis=0, keepdims=True)).astype(jnp.bfloat16)
        att_ref[:, sl] = jax.lax.dot(
            vw_, P, preferred_element_type=jnp.float32).astype(jnp.bfloat16)

    xfb = h_ref[...] + att_ref[...]
    delta = _gelu(jax.lax.dot(pw_ref[...], xfb,
                              preferred_element_type=jnp.float32) + pb_ref[...])
    deltab = delta.astype(jnp.bfloat16) + Xnb
    D = jax.lax.dot(deltab, et_ref[...], preferred_element_type=jnp.float32)
    ml = m_ref[0, s]
    for r in range(WSZ):
        o_ref[0, :, s * WSZ + r, :] = X3[:, r, :] + D[:, r * LB:(r + 1) * LB] * ml


def kernel(feature_map, uncertain_map, norm_w, norm_b, lin0_W, lin0_b,
           qkv0_W, proj_W, proj_b):
    B, C, H, W = feature_map.shape
    nH, nW = H // WSZ, W // WSZ
    nWin = nH * nW
    winsz = WSZ * WSZ
    nWF = int(nWin * FILTER_RATE)
    C2 = lin0_W.shape[1]
    scale = float(C ** (-0.5))

    pool = (np.arange(W)[:, None] // WSZ == np.arange(nW)[None, :])
    pool = jnp.asarray(pool, dtype=jnp.float32)
    poolT = (np.arange(nH)[:, None] == np.arange(H)[None, :] // WSZ)
    poolT = jnp.asarray(poolT, dtype=jnp.float32)
    scores = pl.pallas_call(
        _scores_body,
        out_shape=jax.ShapeDtypeStruct((B, nH, nW), jnp.float32),
    )(uncertain_map, poolT, pool)

    win_score = scores.reshape(B, nWin)
    _, idx = jax.lax.top_k(win_score, nWF)
    mask = jnp.zeros((B, nWin), jnp.float32)
    mask = mask.at[jnp.arange(B)[:, None], idx].set(1.0)
    maskl = jnp.repeat(mask.reshape(B, nH, nW), WSZ, axis=-1)
    maskl = maskl.reshape(B, nH, 1, W)

    G = 4
    LB = W // G
    SP = LB * WSZ

    l_ = np.arange(LB)
    Emat = np.zeros((WSZ * LB, SP), np.float32)
    for r in range(WSZ):
        Emat[r * LB + l_, WSZ * l_ + r] = 1.0
    Eb = jnp.asarray(Emat, dtype=jnp.bfloat16)
    ETb = jnp.asarray(Emat.T.copy(), dtype=jnp.bfloat16)

    l0wt = jnp.asarray(lin0_W.T, dtype=jnp.bfloat16)
    qwt = jnp.asarray(qkv0_W.T, dtype=jnp.bfloat16)
    pwt = jnp.asarray(proj_W.T, dtype=jnp.bfloat16)
    l0b2 = lin0_b.reshape(C2, 1)
    pb2 = proj_b.reshape(C, 1)
    nw2 = norm_w.reshape(C, 1)
    nb2 = norm_b.reshape(C, 1)

    NS = 2
    body = functools.partial(_refine_body, C=C, C2=C2, LB=LB, NS=NS,
                             scale=scale)
    out = pl.pallas_call(
        body,
        grid=(B, nH // NS, G),
        in_specs=[
            pl.BlockSpec((1, NS, 1, LB), lambda b, s, g: (b, s, 0, g)),
            pl.BlockSpec((1, C, NS * WSZ, LB), lambda b, s, g: (b, 0, s, g)),
            pl.BlockSpec((WSZ * LB, SP), lambda b, s, g: (0, 0)),
            pl.BlockSpec((SP, WSZ * LB), lambda b, s, g: (0, 0)),
            pl.BlockSpec((C, 1), lambda b, s, g: (0, 0)),
            pl.BlockSpec((C, 1), lambda b, s, g: (0, 0)),
            pl.BlockSpec((C2, C), lambda b, s, g: (0, 0)),
            pl.BlockSpec((C2, 1), lambda b, s, g: (0, 0)),
            pl.BlockSpec((3 * C2, C2), lambda b, s, g: (0, 0)),
            pl.BlockSpec((C, C2), lambda b, s, g: (0, 0)),
            pl.BlockSpec((C, 1), lambda b, s, g: (0, 0)),
        ],
        out_specs=pl.BlockSpec((1, C, NS * WSZ, LB),
                               lambda b, s, g: (b, 0, s, g)),
        out_shape=jax.ShapeDtypeStruct((B, C, H, W), jnp.float32),
        scratch_shapes=[pltpu.VMEM((C2, SP), jnp.bfloat16),
                        pltpu.VMEM((3 * C2, SP), jnp.bfloat16),
                        pltpu.VMEM((C2, SP), jnp.bfloat16)],
    )(maskl, feature_map, Eb, ETb, nw2, nb2, l0wt, l0b2, qwt, pwt, pb2)
    return out


# speedup vs baseline: 4.0945x; 1.0071x over previous
import functools

import jax
import jax.numpy as jnp
import numpy as np
from jax.experimental import pallas as pl
from jax.experimental.pallas import tpu as pltpu

WSZ = 8
FILTER_RATE = 0.3
EPS = 1e-5


def _gelu(x):
    return x * 0.5 * (1.0 + jax.lax.erf(x * 0.7071067811865476))


def _scores_body(u_ref, poolT_ref, pool_ref, s_ref):
    inv = 1.0 / (WSZ * WSZ)
    hp = jax.lax.Precision.HIGHEST
    for b in range(u_ref.shape[0]):
        t = jax.lax.dot(poolT_ref[...], u_ref[b], precision=hp,
                        preferred_element_type=jnp.float32)
        s_ref[b] = jax.lax.dot(t, pool_ref[...], precision=hp,
                               preferred_element_type=jnp.float32) * inv


def _refine_body(m_ref, x_ref, e_ref, et_ref, nw_ref, nb_ref, l0w_ref,
                 l0b_ref, qw_ref, pw_ref, pb_ref, o_ref,
                 h_ref, qkv_ref, att_ref, *, C, C2, LB, NS, scale):
    for s in range(NS):
        _refine_strip(m_ref, x_ref, e_ref, et_ref, nw_ref, nb_ref, l0w_ref,
                      l0b_ref, qw_ref, pw_ref, pb_ref, o_ref,
                      h_ref, qkv_ref, att_ref, s, C=C, C2=C2, LB=LB,
                      scale=scale)


def _refine_strip(m_ref, x_ref, e_ref, et_ref, nw_ref, nb_ref, l0w_ref,
                  l0b_ref, qw_ref, pw_ref, pb_ref, o_ref,
                  h_ref, qkv_ref, att_ref, s, *, C, C2, LB, scale):
    winsz = WSZ * WSZ
    nwin = LB // WSZ
    SP = LB * WSZ

    X3 = x_ref[0, :, s * WSZ:(s + 1) * WSZ, :]
    X3b = X3.astype(jnp.bfloat16)

    Xw = jnp.zeros((C, SP), jnp.float32)
    for r in range(WSZ):
        Xw = Xw + jax.lax.dot(X3b[:, r, :], e_ref[r * LB:(r + 1) * LB, :],
                              preferred_element_type=jnp.float32)

    mu = jnp.mean(Xw, axis=0, keepdims=True)
    var = jnp.mean(Xw * Xw, axis=0, keepdims=True) - mu * mu
    Xnb = ((Xw - mu) * jax.lax.rsqrt(var + EPS) * nw_ref[...]
           + nb_ref[...]).astype(jnp.bfloat16)

    h_ref[...] = _gelu(
        jax.lax.dot(l0w_ref[...], Xnb, preferred_element_type=jnp.float32)
        + l0b_ref[...]).astype(jnp.bfloat16)
    qkv_ref[...] = jax.lax.dot(
        qw_ref[...], h_ref[...],
        preferred_element_type=jnp.float32).astype(jnp.bfloat16)

    dn = (((0,), (0,)), ((), ()))
    for w in range(nwin):
        sl = slice(w * winsz, (w + 1) * winsz)
        qw_ = qkv_ref[0:C2, sl]
        kw_ = qkv_ref[C2:2 * C2, sl]
        vw_ = qkv_ref[2 * C2:3 * C2, sl]
        S = jax.lax.dot_general(kw_, qw_, dn,
                                preferred_element_type=jnp.float32) * scale
        S = S - jnp.max(S, axis=0, keepdims=True)
        E = jnp.exp(S)
        P = (E / jnp.sum(E, axis=0, keepdims=True)).astype(jnp.bfloat16)
        att_ref[:, sl] = jax.lax.dot(
            vw_, P, preferred_element_type=jnp.float32).astype(jnp.bfloat16)

    xfb = h_ref[...] + att_ref[...]
    delta = _gelu(jax.lax.dot(pw_ref[...], xfb,
                              preferred_element_type=jnp.float32) + pb_ref[...])
    deltab = delta.astype(jnp.bfloat16) + Xnb
    D = jax.lax.dot(deltab, et_ref[...], preferred_element_type=jnp.float32)
    ml = m_ref[0, s]
    for r in range(WSZ):
        o_ref[0, :, s * WSZ + r, :] = X3[:, r, :] + D[:, r * LB:(r + 1) * LB] * ml


def kernel(feature_map, uncertain_map, norm_w, norm_b, lin0_W, lin0_b,
           qkv0_W, proj_W, proj_b):
    B, C, H, W = feature_map.shape
    nH, nW = H // WSZ, W // WSZ
    nWin = nH * nW
    winsz = WSZ * WSZ
    nWF = int(nWin * FILTER_RATE)
    C2 = lin0_W.shape[1]
    scale = float(C ** (-0.5))

    pool = (np.arange(W)[:, None] // WSZ == np.arange(nW)[None, :])
    pool = jnp.asarray(pool, dtype=jnp.float32)
    poolT = (np.arange(nH)[:, None] == np.arange(H)[None, :] // WSZ)
    poolT = jnp.asarray(poolT, dtype=jnp.float32)
    scores = pl.pallas_call(
        _scores_body,
        out_shape=jax.ShapeDtypeStruct((B, nH, nW), jnp.float32),
    )(uncertain_map, poolT, pool)

    win_score = scores.reshape(B, nWin)
    vals, _ = jax.lax.top_k(win_score, nWF)
    thr = vals[:, nWF - 1:nWF]
    gt = win_score > thr
    tie = win_score == thr
    need = nWF - jnp.sum(gt, axis=-1, keepdims=True)
    tie_rank = jnp.cumsum(tie.astype(jnp.int32), axis=-1)
    mask = (gt | (tie & (tie_rank <= need))).astype(jnp.float32)
    maskl = jnp.repeat(mask.reshape(B, nH, nW), WSZ, axis=-1)
    maskl = maskl.reshape(B, nH, 1, W)

    G = 4
    LB = W // G
    SP = LB * WSZ

    l_ = np.arange(LB)
    Emat = np.zeros((WSZ * LB, SP), np.float32)
    for r in range(WSZ):
        Emat[r * LB + l_, WSZ * l_ + r] = 1.0
    Eb = jnp.asarray(Emat, dtype=jnp.bfloat16)
    ETb = jnp.asarray(Emat.T.copy(), dtype=jnp.bfloat16)

    l0wt = jnp.asarray(lin0_W.T, dtype=jnp.bfloat16)
    qwt = jnp.asarray(qkv0_W.T, dtype=jnp.bfloat16)
    pwt = jnp.asarray(proj_W.T, dtype=jnp.bfloat16)
    l0b2 = lin0_b.reshape(C2, 1)
    pb2 = proj_b.reshape(C, 1)
    nw2 = norm_w.reshape(C, 1)
    nb2 = norm_b.reshape(C, 1)

    NS = 2
    body = functools.partial(_refine_body, C=C, C2=C2, LB=LB, NS=NS,
                             scale=scale)
    out = pl.pallas_call(
        body,
        grid=(B, nH // NS, G),
        in_specs=[
            pl.BlockSpec((1, NS, 1, LB), lambda b, s, g: (b, s, 0, g)),
            pl.BlockSpec((1, C, NS * WSZ, LB), lambda b, s, g: (b, 0, s, g)),
            pl.BlockSpec((WSZ * LB, SP), lambda b, s, g: (0, 0)),
            pl.BlockSpec((SP, WSZ * LB), lambda b, s, g: (0, 0)),
            pl.BlockSpec((C, 1), lambda b, s, g: (0, 0)),
            pl.BlockSpec((C, 1), lambda b, s, g: (0, 0)),
            pl.BlockSpec((C2, C), lambda b, s, g: (0, 0)),
            pl.BlockSpec((C2, 1), lambda b, s, g: (0, 0)),
            pl.BlockSpec((3 * C2, C2), lambda b, s, g: (0, 0)),
            pl.BlockSpec((C, C2), lambda b, s, g: (0, 0)),
            pl.BlockSpec((C, 1), lambda b, s, g: (0, 0)),
        ],
        out_specs=pl.BlockSpec((1, C, NS * WSZ, LB),
                               lambda b, s, g: (b, 0, s, g)),
        out_shape=jax.ShapeDtypeStruct((B, C, H, W), jnp.float32),
        scratch_shapes=[pltpu.VMEM((C2, SP), jnp.bfloat16),
                        pltpu.VMEM((3 * C2, SP), jnp.bfloat16),
                        pltpu.VMEM((C2, SP), jnp.bfloat16)],
    )(maskl, feature_map, Eb, ETb, nw2, nb2, l0wt, l0b2, qwt, pwt, pb2)
    return out


# speedup vs baseline: 4.1232x; 1.0070x over previous
import functools

import jax
import jax.numpy as jnp
import numpy as np
from jax.experimental import pallas as pl
from jax.experimental.pallas import tpu as pltpu

WSZ = 8
FILTER_RATE = 0.3
EPS = 1e-5


def _gelu(x):
    return x * 0.5 * (1.0 + jax.lax.erf(x * 0.7071067811865476))


def _scores_body(u_ref, poolT_ref, pool_ref, s_ref):
    inv = 1.0 / (WSZ * WSZ)
    hp = jax.lax.Precision.HIGHEST
    for b in range(u_ref.shape[0]):
        t = jax.lax.dot(poolT_ref[...], u_ref[b], precision=hp,
                        preferred_element_type=jnp.float32)
        s_ref[b] = jax.lax.dot(t, pool_ref[...], precision=hp,
                               preferred_element_type=jnp.float32) * inv


def _refine_body(m_ref, x_ref, e_ref, et_ref, nw_ref, nb_ref, l0w_ref,
                 l0b_ref, qw_ref, pw_ref, pb_ref, o_ref,
                 h_ref, qkv_ref, att_ref, *, C, C2, LB, NS, scale):
    for s in range(NS):
        _refine_strip(m_ref, x_ref, e_ref, et_ref, nw_ref, nb_ref, l0w_ref,
                      l0b_ref, qw_ref, pw_ref, pb_ref, o_ref,
                      h_ref, qkv_ref, att_ref, s, C=C, C2=C2, LB=LB,
                      scale=scale)


def _refine_strip(m_ref, x_ref, e_ref, et_ref, nw_ref, nb_ref, l0w_ref,
                  l0b_ref, qw_ref, pw_ref, pb_ref, o_ref,
                  h_ref, qkv_ref, att_ref, s, *, C, C2, LB, scale):
    winsz = WSZ * WSZ
    nwin = LB // WSZ
    SP = LB * WSZ

    X3 = x_ref[0, :, s * WSZ:(s + 1) * WSZ, :]
    X3b = X3.astype(jnp.bfloat16)

    Xw = jnp.zeros((C, SP), jnp.float32)
    for r in range(WSZ):
        Xw = Xw + jax.lax.dot(X3b[:, r, :], e_ref[r * LB:(r + 1) * LB, :],
                              preferred_element_type=jnp.float32)

    mu = jnp.mean(Xw, axis=0, keepdims=True)
    var = jnp.mean(Xw * Xw, axis=0, keepdims=True) - mu * mu
    Xnb = ((Xw - mu) * jax.lax.rsqrt(var + EPS) * nw_ref[...]
           + nb_ref[...]).astype(jnp.bfloat16)

    h_ref[...] = _gelu(
        jax.lax.dot(l0w_ref[...], Xnb, preferred_element_type=jnp.float32)
        + l0b_ref[...]).astype(jnp.bfloat16)
    qkv_ref[...] = jax.lax.dot(
        qw_ref[...], h_ref[...],
        preferred_element_type=jnp.float32).astype(jnp.bfloat16)

    dn = (((0,), (0,)), ((), ()))
    for w in range(nwin):
        sl = slice(w * winsz, (w + 1) * winsz)
        qw_ = qkv_ref[0:C2, sl]
        kw_ = qkv_ref[C2:2 * C2, sl]
        vw_ = qkv_ref[2 * C2:3 * C2, sl]
        S = jax.lax.dot_general(kw_, qw_, dn,
                                preferred_element_type=jnp.float32) * scale
        S = S - jnp.max(S, axis=0, keepdims=True)
        E = jnp.exp(S)
        P = (E / jnp.sum(E, axis=0, keepdims=True)).astype(jnp.bfloat16)
        att_ref[:, sl] = jax.lax.dot(
            vw_, P, preferred_element_type=jnp.float32).astype(jnp.bfloat16)

    xfb = h_ref[...] + att_ref[...]
    delta = _gelu(jax.lax.dot(pw_ref[...], xfb,
                              preferred_element_type=jnp.float32) + pb_ref[...])
    deltab = delta.astype(jnp.bfloat16) + Xnb
    D = jax.lax.dot(deltab, et_ref[...], preferred_element_type=jnp.float32)
    ml = m_ref[0, s]
    for r in range(WSZ):
        o_ref[0, :, s * WSZ + r, :] = X3[:, r, :] + D[:, r * LB:(r + 1) * LB] * ml


def kernel(feature_map, uncertain_map, norm_w, norm_b, lin0_W, lin0_b,
           qkv0_W, proj_W, proj_b):
    B, C, H, W = feature_map.shape
    nH, nW = H // WSZ, W // WSZ
    nWin = nH * nW
    winsz = WSZ * WSZ
    nWF = int(nWin * FILTER_RATE)
    C2 = lin0_W.shape[1]
    scale = float(C ** (-0.5))

    pool = (np.arange(W)[:, None] // WSZ == np.arange(nW)[None, :])
    pool = jnp.asarray(pool, dtype=jnp.float32)
    poolT = (np.arange(nH)[:, None] == np.arange(H)[None, :] // WSZ)
    poolT = jnp.asarray(poolT, dtype=jnp.float32)
    scores = pl.pallas_call(
        _scores_body,
        out_shape=jax.ShapeDtypeStruct((B, nH, nW), jnp.float32),
    )(uncertain_map, poolT, pool)

    win_score = scores.reshape(B, nWin)
    vals, _ = jax.lax.top_k(win_score, nWF)
    thr = vals[:, nWF - 1:nWF]
    gt = win_score > thr
    tie = win_score == thr
    need = nWF - jnp.sum(gt, axis=-1, keepdims=True)
    tie_rank = jnp.cumsum(tie.astype(jnp.int32), axis=-1)
    mask = (gt | (tie & (tie_rank <= need))).astype(jnp.float32)
    maskl = jnp.repeat(mask.reshape(B, nH, nW), WSZ, axis=-1)
    maskl = maskl.reshape(B, nH, 1, W)

    G = 4
    LB = W // G
    SP = LB * WSZ

    l_ = np.arange(LB)
    Emat = np.zeros((WSZ * LB, SP), np.float32)
    for r in range(WSZ):
        Emat[r * LB + l_, WSZ * l_ + r] = 1.0
    Eb = jnp.asarray(Emat, dtype=jnp.bfloat16)
    ETb = jnp.asarray(Emat.T.copy(), dtype=jnp.bfloat16)

    l0wt = jnp.asarray(lin0_W.T, dtype=jnp.bfloat16)
    qwt = jnp.asarray(qkv0_W.T, dtype=jnp.bfloat16)
    pwt = jnp.asarray(proj_W.T, dtype=jnp.bfloat16)
    l0b2 = lin0_b.reshape(C2, 1)
    pb2 = proj_b.reshape(C, 1)
    nw2 = norm_w.reshape(C, 1)
    nb2 = norm_b.reshape(C, 1)

    NS = 4
    body = functools.partial(_refine_body, C=C, C2=C2, LB=LB, NS=NS,
                             scale=scale)
    out = pl.pallas_call(
        body,
        grid=(B, nH // NS, G),
        in_specs=[
            pl.BlockSpec((1, NS, 1, LB), lambda b, s, g: (b, s, 0, g)),
            pl.BlockSpec((1, C, NS * WSZ, LB), lambda b, s, g: (b, 0, s, g)),
            pl.BlockSpec((WSZ * LB, SP), lambda b, s, g: (0, 0)),
            pl.BlockSpec((SP, WSZ * LB), lambda b, s, g: (0, 0)),
            pl.BlockSpec((C, 1), lambda b, s, g: (0, 0)),
            pl.BlockSpec((C, 1), lambda b, s, g: (0, 0)),
            pl.BlockSpec((C2, C), lambda b, s, g: (0, 0)),
            pl.BlockSpec((C2, 1), lambda b, s, g: (0, 0)),
            pl.BlockSpec((3 * C2, C2), lambda b, s, g: (0, 0)),
            pl.BlockSpec((C, C2), lambda b, s, g: (0, 0)),
            pl.BlockSpec((C, 1), lambda b, s, g: (0, 0)),
        ],
        out_specs=pl.BlockSpec((1, C, NS * WSZ, LB),
                               lambda b, s, g: (b, 0, s, g)),
        out_shape=jax.ShapeDtypeStruct((B, C, H, W), jnp.float32),
        scratch_shapes=[pltpu.VMEM((C2, SP), jnp.bfloat16),
                        pltpu.VMEM((3 * C2, SP), jnp.bfloat16),
                        pltpu.VMEM((C2, SP), jnp.bfloat16)],
    )(maskl, feature_map, Eb, ETb, nw2, nb2, l0wt, l0b2, qwt, pwt, pb2)
    return out
